# baseline (speedup 1.0000x reference)
import functools

import jax
import jax.numpy as jnp
from jax import lax
from jax.experimental import pallas as pl
from jax.experimental.pallas import tpu as pltpu
from jax.experimental.pallas import tpu_sc as plsc

N = 10000
E = 320000
L = 128
STEPS = 5
TW = 5
TD = 3

NWORK = 32
CHUNK = 128
CH_PER_W = 80
E_PAD = NWORK * CH_PER_W * CHUNK
N_PAD = 10240

BLK_E = 2048
BLK_N = 1280

_F32 = jnp.float32


def _ln(h, g, bb):
    mu = jnp.mean(h, axis=-1, keepdims=True)
    var = jnp.mean((h - mu) ** 2, axis=-1, keepdims=True)
    return (h - mu) / jnp.sqrt(var + 1e-5) * g + bb


def _sc_mesh():
    return plsc.VectorSubcoreMesh(core_axis_name="c", subcore_axis_name="s",
                                  num_cores=2, num_subcores=16)


CH_T = E_PAD // 16 // CHUNK
NBUF = 4


@jax.jit
def _sc_gather2(tabA, tabB, idxS16, idxR16):
    rows_sub = N_PAD // 16

    @functools.partial(
        pl.kernel,
        out_type=jax.ShapeDtypeStruct((2, E_PAD, L), _F32),
        mesh=_sc_mesh(),
        scratch_types=[
            pltpu.VMEM((CH_T, CHUNK), jnp.int32),
            [pltpu.VMEM((CHUNK, L), _F32)] * NBUF,
            [pltpu.SemaphoreType.DMA] * NBUF,
            [pltpu.SemaphoreType.DMA] * NBUF,
        ],
    )
    def k(tabA_h, tabB_h, iS_h, iR_h, out_h, idx_v, bufs, semG, semO):
        c = lax.axis_index("c")
        s = lax.axis_index("s")

        @pl.when(c == 0)
        def _():
            pltpu.sync_copy(iS_h.at[s], idx_v)

        @pl.when(c == 1)
        def _():
            pltpu.sync_copy(iR_h.at[s], idx_v)

        base = s * (CH_T * CHUNK)

        def fire(j, b):
            @pl.when(c == 0)
            def _():
                pltpu.async_copy(tabA_h.at[idx_v.at[j]], bufs[b], semG[b])

            @pl.when(c == 1)
            def _():
                pltpu.async_copy(tabB_h.at[idx_v.at[j]], bufs[b], semG[b])

        def wait_gather(j, b):
            @pl.when(c == 0)
            def _():
                pltpu.make_async_copy(tabA_h.at[idx_v.at[j]], bufs[b],
                                      semG[b]).wait()

            @pl.when(c == 1)
            def _():
                pltpu.make_async_copy(tabB_h.at[idx_v.at[j]], bufs[b],
                                      semG[b]).wait()

        for b in range(NBUF):
            fire(b, b)

        def body(jj, carry):
            for b in range(NBUF):
                j = jj * NBUF + b
                wait_gather(j, b)
                dst = out_h.at[c, pl.ds(base + j * CHUNK, CHUNK)]
                pltpu.async_copy(bufs[b], dst, semO[b])
                pltpu.make_async_copy(bufs[b], dst, semO[b]).wait()

                @pl.when(j + NBUF < CH_T)
                def _():
                    fire(j + NBUF, b)
            return carry

        lax.fori_loop(0, CH_T // NBUF, body, 0)

    return k(tabA, tabB, idxS16, idxR16)


@jax.jit
def _sc_scatter_add(vals, idxR, zrows):
    rows_sub = N_PAD // 16

    @functools.partial(
        pl.kernel,
        out_type=jax.ShapeDtypeStruct((2, N_PAD, L), _F32),
        mesh=_sc_mesh(),
        scratch_types=[
            pltpu.VMEM((CH_PER_W, CHUNK), jnp.int32),
            [pltpu.VMEM((CHUNK, L), _F32)] * 2,
            pltpu.VMEM_SHARED((N_PAD, L), _F32),
            [pltpu.SemaphoreType.DMA] * 2,
            [pltpu.SemaphoreType.DMA] * 2,
        ],
    )
    def k(vals_h, idxR_h, z_h, out_h, idx_v, buf, acc, semL, semS):
        c = lax.axis_index("c")
        s = lax.axis_index("s")
        wid = s * 2 + c
        pltpu.sync_copy(idxR_h.at[wid], idx_v)
        pltpu.sync_copy(z_h.at[pl.ds(s * rows_sub, rows_sub)],
                        acc.at[pl.ds(s * rows_sub, rows_sub)])
        plsc.subcore_barrier()
        base = wid * (CH_PER_W * CHUNK)

        def load(j, b):
            pltpu.async_copy(vals_h.at[pl.ds(base + j * CHUNK, CHUNK)],
                             buf[b], semL[b])

        load(0, 0)
        load(1, 1)

        def body(jj, carry):
            for b in range(2):
                j = jj * 2 + b
                pltpu.make_async_copy(
                    vals_h.at[pl.ds(base + j * CHUNK, CHUNK)], buf[b],
                    semL[b]).wait()
                pltpu.async_copy(buf[b], acc.at[idx_v.at[j]], semS[b],
                                 add=True).wait()

                @pl.when(j + 2 < CH_PER_W)
                def _():
                    load(j + 2, b)
            return carry

        lax.fori_loop(0, CH_PER_W // 2, body, 0)
        plsc.subcore_barrier()
        pltpu.sync_copy(acc.at[pl.ds(s * rows_sub, rows_sub)],
                        out_h.at[c, pl.ds(s * rows_sub, rows_sub)])

    return k(vals, idxR, zrows)


def _edge_encode_body(dA_ref, dB_ref, W1d, w1n, b1, W2, b2, g, bb, el0):
    d = dA_ref[0] + dB_ref[0]
    m = (lax.broadcasted_iota(jnp.int32, (1, L), 1) < 3).astype(_F32)
    norm = jnp.sqrt(jnp.sum((d * m) ** 2, axis=-1, keepdims=True))
    x = jnp.dot(d, W1d[...], preferred_element_type=_F32) + norm * w1n[...] + b1[...]
    h1 = jnp.maximum(x, 0.0)
    h2 = jnp.maximum(jnp.dot(h1, W2[...], preferred_element_type=_F32) + b2[...], 0.0)
    el0[...] = _ln(h2, g[...], bb[...])


def _node_encode_body(nf_ref, W1p, b1, W2, b2, g, bb, WnA, WnB, nl0, A0, B0):
    x = jnp.dot(nf_ref[...], W1p[...], preferred_element_type=_F32) + b1[...]
    h1 = jnp.maximum(x, 0.0)
    h2 = jnp.maximum(jnp.dot(h1, W2[...], preferred_element_type=_F32) + b2[...], 0.0)
    nl = _ln(h2, g[...], bb[...])
    nl0[...] = nl
    A0[...] = jnp.dot(nl, WnA[...], preferred_element_type=_F32)
    B0[...] = jnp.dot(nl, WnB[...], preferred_element_type=_F32)


def _edge_step_body(el_ref, GA_ref, GB_ref, W1c, b1, W2, b2, g, bb, elo, nel):
    el = el_ref[...]
    x = (jnp.dot(el, W1c[...], preferred_element_type=_F32)
         + GA_ref[0] + GB_ref[0] + b1[...])
    h1 = jnp.maximum(x, 0.0)
    h2 = jnp.maximum(jnp.dot(h1, W2[...], preferred_element_type=_F32) + b2[...], 0.0)
    out = _ln(h2, g[...], bb[...])
    nel[...] = out
    elo[...] = el + out


def _node_step_body(nl_ref, a0_ref, a1_ref, Wa, Wb, b1, W2, b2, g, bb,
                    WnA, WnB, nlo, A, B):
    nl = nl_ref[...]
    agg = a0_ref[...] + a1_ref[...]
    x = (jnp.dot(nl, Wa[...], preferred_element_type=_F32)
         + jnp.dot(agg, Wb[...], preferred_element_type=_F32) + b1[...])
    h1 = jnp.maximum(x, 0.0)
    h2 = jnp.maximum(jnp.dot(h1, W2[...], preferred_element_type=_F32) + b2[...], 0.0)
    out = nl + _ln(h2, g[...], bb[...])
    nlo[...] = out
    A[...] = jnp.dot(out, WnA[...], preferred_element_type=_F32)
    B[...] = jnp.dot(out, WnB[...], preferred_element_type=_F32)


def _node_last_body(nl_ref, a0_ref, a1_ref, Wa, Wb, b1, W2, b2, g, bb,
                    dW1, db1, dW2, db2, dtp, dec):
    nl = nl_ref[...]
    agg = a0_ref[...] + a1_ref[...]
    x = (jnp.dot(nl, Wa[...], preferred_element_type=_F32)
         + jnp.dot(agg, Wb[...], preferred_element_type=_F32) + b1[...])
    h1 = jnp.maximum(x, 0.0)
    h2 = jnp.maximum(jnp.dot(h1, W2[...], preferred_element_type=_F32) + b2[...], 0.0)
    out = nl + _ln(h2, g[...], bb[...])
    h = jnp.dot(out, dW1[...], preferred_element_type=_F32) + db1[...]
    h = h * jax.nn.sigmoid(h)
    dec[...] = (jnp.dot(h, dW2[...], preferred_element_type=_F32) + db2[...]) * dtp[...]


def _row_spec(blk, w):
    return pl.BlockSpec((blk, w), lambda i: (i, 0))


def _full_spec(shape):
    nd = len(shape)
    return pl.BlockSpec(shape, lambda i: (0,) * nd)


def _part_spec(blk, part):
    return pl.BlockSpec((1, blk, L), lambda i, _p=part: (_p, i, 0))


def _tc_call(body, grid, in_arrays, blocked, out_shapes, out_blocked):
    in_specs = []
    for a, b in zip(in_arrays, blocked):
        if isinstance(b, pl.BlockSpec):
            in_specs.append(b)
        else:
            in_specs.append(_row_spec(*b) if b is not None else _full_spec(a.shape))
    out_specs = [_row_spec(*b) if b is not None else _full_spec(s.shape)
                 for s, b in zip(out_shapes, out_blocked)]
    return pl.pallas_call(
        body,
        grid=(grid,),
        in_specs=in_specs,
        out_specs=out_specs if len(out_specs) > 1 else out_specs[0],
        out_shape=out_shapes if len(out_shapes) > 1 else out_shapes[0],
    )(*in_arrays)


def _r1(v):
    return v.reshape(1, L)


def kernel(mesh_pos, node_type, u, load, senders, receivers,
           ne_W1, ne_b1, ne_W2, ne_b2, ne_g, ne_bb,
           ee_W1, ee_b1, ee_W2, ee_b2, ee_g, ee_bb,
           be_W1, be_b1, be_W2, be_b2, be_g, be_bb,
           bn_W1, bn_b1, bn_W2, bn_b2, bn_g, bn_bb,
           dec_W1, dec_b1, dec_W2, dec_b2):
    mesh_pos = mesh_pos[0]
    node_type = node_type[0]
    u = u[0]
    load = load[0]

    pad_e = E_PAD - E
    s_pad = jnp.concatenate([senders.astype(jnp.int32),
                             jnp.full((pad_e,), N, jnp.int32)])
    r_pad = jnp.concatenate([receivers.astype(jnp.int32),
                             jnp.full((pad_e,), N, jnp.int32)])
    s16 = s_pad.reshape(16, CH_T, CHUNK)
    r16 = r_pad.reshape(16, CH_T, CHUNK)
    r_w = r_pad.reshape(NWORK, CH_PER_W, CHUNK)

    T = jnp.zeros((N_PAD, L), _F32)
    T = T.at[:N, 0:3].set(mesh_pos).at[:N, 3:6].set(u)
    Tn = -T

    dparts = _sc_gather2(T, Tn, s16, r16)

    W1d = jnp.zeros((L, L), _F32).at[0:6, :].set(ee_W1[0:6, :])
    w1n = ee_W1[6:7, :]
    el = _tc_call(
        _edge_encode_body, E_PAD // BLK_E,
        [dparts, dparts, W1d, w1n, _r1(ee_b1), ee_W2, _r1(ee_b2), _r1(ee_g),
         _r1(ee_bb)],
        [_part_spec(BLK_E, 0), _part_spec(BLK_E, 1),
         None, None, None, None, None, None, None],
        [jax.ShapeDtypeStruct((E_PAD, L), _F32)], [(BLK_E, L)])

    nf = jnp.zeros((N_PAD, 16), _F32)
    nf = nf.at[:N, 0:3].set(u).at[:N, 3:6].set(load).at[:N, 6:15].set(node_type)
    neW1p = jnp.zeros((16, L), _F32).at[0:15, :].set(ne_W1)
    nl, A, B = _tc_call(
        _node_encode_body, N_PAD // BLK_N,
        [nf, neW1p, _r1(ne_b1), ne_W2, _r1(ne_b2), _r1(ne_g), _r1(ne_bb),
         be_W1[0, 0:L, :], be_W1[0, L:2 * L, :]],
        [(BLK_N, 16)] + [None] * 8,
        [jax.ShapeDtypeStruct((N_PAD, L), _F32)] * 3,
        [(BLK_N, L)] * 3)

    zeros_n = jnp.zeros((N_PAD, L), _F32)

    for i in range(STEPS):
        Gp = _sc_gather2(A, B, s16, r16)
        el, new_el = _tc_call(
            _edge_step_body, E_PAD // BLK_E,
            [el, Gp, Gp, be_W1[i, 2 * L:3 * L, :], _r1(be_b1[i]), be_W2[i],
             _r1(be_b2[i]), _r1(be_g[i]), _r1(be_bb[i])],
            [(BLK_E, L), _part_spec(BLK_E, 0), _part_spec(BLK_E, 1),
             None, None, None, None, None, None],
            [jax.ShapeDtypeStruct((E_PAD, L), _F32)] * 2,
            [(BLK_E, L)] * 2)

        parts = _sc_scatter_add(new_el, r_w, zeros_n)
        ag0, ag1 = parts[0], parts[1]

        if i < STEPS - 1:
            nl, A, B = _tc_call(
                _node_step_body, N_PAD // BLK_N,
                [nl, ag0, ag1, bn_W1[i, 0:L, :], bn_W1[i, L:2 * L, :],
                 _r1(bn_b1[i]), bn_W2[i], _r1(bn_b2[i]), _r1(bn_g[i]),
                 _r1(bn_bb[i]), be_W1[i + 1, 0:L, :], be_W1[i + 1, L:2 * L, :]],
                [(BLK_N, L)] * 3 + [None] * 9,
                [jax.ShapeDtypeStruct((N_PAD, L), _F32)] * 3,
                [(BLK_N, L)] * 3)
        else:
            dW1p = jnp.zeros((L, L), _F32).at[:, 0:8].set(dec_W1)
            db1p = jnp.zeros((1, L), _F32).at[0, 0:8].set(dec_b1)
            dW2p = jnp.zeros((L, L), _F32).at[0:8, 0:TD * TW].set(dec_W2)
            db2p = jnp.zeros((1, L), _F32).at[0, 0:TD * TW].set(dec_b2)
            dt = jnp.repeat(jnp.arange(1, TW + 1, dtype=_F32), TD)
            dtp = jnp.zeros((1, L), _F32).at[0, 0:TD * TW].set(dt)
            dec = _tc_call(
                _node_last_body, N_PAD // BLK_N,
                [nl, ag0, ag1, bn_W1[i, 0:L, :], bn_W1[i, L:2 * L, :],
                 _r1(bn_b1[i]), bn_W2[i], _r1(bn_b2[i]), _r1(bn_g[i]),
                 _r1(bn_bb[i]), dW1p, db1p, dW2p, db2p, dtp],
                [(BLK_N, L)] * 3 + [None] * 12,
                [jax.ShapeDtypeStruct((N_PAD, L), _F32)], [(BLK_N, L)])

    return dec[:N, 0:TD * TW].reshape(N, TW, TD).transpose(1, 0, 2)


# speedup vs baseline: 1.5358x; 1.5358x over previous
import functools

import jax
import jax.numpy as jnp
from jax import lax
from jax.experimental import pallas as pl
from jax.experimental.pallas import tpu as pltpu
from jax.experimental.pallas import tpu_sc as plsc

N = 10000
E = 320000
L = 128
STEPS = 5
TW = 5
TD = 3

NWORK = 32
CHUNK = 128
CH_PER_W = 80
E_PAD = NWORK * CH_PER_W * CHUNK
N_PAD = 10240

BLK_E = 2048
BLK_N = 1280

_F32 = jnp.float32
_BF16 = jnp.bfloat16


def _ln(h, g, bb):
    mu = jnp.mean(h, axis=-1, keepdims=True)
    var = jnp.mean((h - mu) ** 2, axis=-1, keepdims=True)
    return (h - mu) / jnp.sqrt(var + 1e-5) * g + bb


def _sc_mesh():
    return plsc.VectorSubcoreMesh(core_axis_name="c", subcore_axis_name="s",
                                  num_cores=2, num_subcores=16)


CH_T = E_PAD // 16 // CHUNK
NBUF = 4
N_PK = N_PAD // 2


def _pack2(x_f32):
    ub = lax.bitcast_convert_type(x_f32.astype(_BF16), jnp.uint16)
    ub = ub.astype(jnp.uint32).reshape(N_PK, 2, L)
    return lax.bitcast_convert_type(ub[:, 0, :] | (ub[:, 1, :] << 16),
                                    jnp.int32)


def _unpack2(p_i32, parity):
    even = lax.bitcast_convert_type(p_i32 << 16, _F32)
    odd = lax.bitcast_convert_type(
        p_i32 & jnp.int32(-65536), _F32)
    return jnp.where(parity > 0.5, odd, even)


@jax.jit
def _sc_gather2(tabA, tabB, idxS16, idxR16):
    rows_sub = N_PK // 16

    @functools.partial(
        pl.kernel,
        out_type=jax.ShapeDtypeStruct((2, E_PAD, L), jnp.int32),
        mesh=_sc_mesh(),
        scratch_types=[
            pltpu.VMEM((CH_T, CHUNK), jnp.int32),
            [pltpu.VMEM((CHUNK, L), jnp.int32)] * NBUF,
            pltpu.VMEM_SHARED((N_PK, L), jnp.int32),
            [pltpu.SemaphoreType.DMA] * NBUF,
            [pltpu.SemaphoreType.DMA] * NBUF,
        ],
    )
    def k(tabA_h, tabB_h, iS_h, iR_h, out_h, idx_v, bufs, spm, semG, semO):
        c = lax.axis_index("c")
        s = lax.axis_index("s")
        sl_sub = pl.ds(s * rows_sub, rows_sub)

        @pl.when(c == 0)
        def _():
            pltpu.sync_copy(iS_h.at[s], idx_v)
            pltpu.sync_copy(tabA_h.at[sl_sub], spm.at[sl_sub])

        @pl.when(c == 1)
        def _():
            pltpu.sync_copy(iR_h.at[s], idx_v)
            pltpu.sync_copy(tabB_h.at[sl_sub], spm.at[sl_sub])

        plsc.subcore_barrier()
        base = s * (CH_T * CHUNK)

        def fire(j, b):
            pltpu.async_copy(spm.at[idx_v.at[j]], bufs[b], semG[b])

        for b in range(NBUF):
            fire(b, b)

        def body(jj, carry):
            for b in range(NBUF):
                j = jj * NBUF + b
                pltpu.make_async_copy(spm.at[idx_v.at[j]], bufs[b],
                                      semG[b]).wait()
                dst = out_h.at[c, pl.ds(base + j * CHUNK, CHUNK)]
                pltpu.async_copy(bufs[b], dst, semO[b])
                pltpu.make_async_copy(bufs[b], dst, semO[b]).wait()

                @pl.when(j + NBUF < CH_T)
                def _():
                    fire(j + NBUF, b)
            return carry

        lax.fori_loop(0, CH_T // NBUF, body, 0)

    return k(tabA, tabB, idxS16, idxR16)


@jax.jit
def _sc_scatter_add(vals, idxR, zrows):
    rows_sub = N_PAD // 16

    @functools.partial(
        pl.kernel,
        out_type=jax.ShapeDtypeStruct((2, N_PAD, L), _F32),
        mesh=_sc_mesh(),
        scratch_types=[
            pltpu.VMEM((CH_PER_W, CHUNK), jnp.int32),
            [pltpu.VMEM((CHUNK, L), _F32)] * 2,
            pltpu.VMEM_SHARED((N_PAD, L), _F32),
            [pltpu.SemaphoreType.DMA] * 2,
            [pltpu.SemaphoreType.DMA] * 2,
        ],
    )
    def k(vals_h, idxR_h, z_h, out_h, idx_v, buf, acc, semL, semS):
        c = lax.axis_index("c")
        s = lax.axis_index("s")
        wid = s * 2 + c
        pltpu.sync_copy(idxR_h.at[wid], idx_v)
        pltpu.sync_copy(z_h.at[pl.ds(s * rows_sub, rows_sub)],
                        acc.at[pl.ds(s * rows_sub, rows_sub)])
        plsc.subcore_barrier()
        base = wid * (CH_PER_W * CHUNK)

        def load(j, b):
            pltpu.async_copy(vals_h.at[pl.ds(base + j * CHUNK, CHUNK)],
                             buf[b], semL[b])

        load(0, 0)
        load(1, 1)

        def body(jj, carry):
            for b in range(2):
                j = jj * 2 + b
                pltpu.make_async_copy(
                    vals_h.at[pl.ds(base + j * CHUNK, CHUNK)], buf[b],
                    semL[b]).wait()
                pltpu.async_copy(buf[b], acc.at[idx_v.at[j]], semS[b],
                                 add=True).wait()

                @pl.when(j + 2 < CH_PER_W)
                def _():
                    load(j + 2, b)
            return carry

        lax.fori_loop(0, CH_PER_W // 2, body, 0)
        plsc.subcore_barrier()
        pltpu.sync_copy(acc.at[pl.ds(s * rows_sub, rows_sub)],
                        out_h.at[c, pl.ds(s * rows_sub, rows_sub)])

    return k(vals, idxR, zrows)


def _edge_encode_body(dA_ref, dB_ref, pS_ref, pR_ref, W1d, w1n, b1, W2, b2,
                      g, bb, el0):
    d = _unpack2(dA_ref[0], pS_ref[...]) + _unpack2(dB_ref[0], pR_ref[...])
    m = (lax.broadcasted_iota(jnp.int32, (1, L), 1) < 3).astype(_F32)
    norm = jnp.sqrt(jnp.sum((d * m) ** 2, axis=-1, keepdims=True))
    x = jnp.dot(d, W1d[...], preferred_element_type=_F32) + norm * w1n[...] + b1[...]
    h1 = jnp.maximum(x, 0.0)
    h2 = jnp.maximum(jnp.dot(h1, W2[...], preferred_element_type=_F32) + b2[...], 0.0)
    el0[...] = _ln(h2, g[...], bb[...])


def _node_encode_body(nf_ref, W1p, b1, W2, b2, g, bb, WnA, WnB, nl0, A0, B0):
    x = jnp.dot(nf_ref[...], W1p[...], preferred_element_type=_F32) + b1[...]
    h1 = jnp.maximum(x, 0.0)
    h2 = jnp.maximum(jnp.dot(h1, W2[...], preferred_element_type=_F32) + b2[...], 0.0)
    nl = _ln(h2, g[...], bb[...])
    nl0[...] = nl
    A0[...] = jnp.dot(nl, WnA[...], preferred_element_type=_F32)
    B0[...] = jnp.dot(nl, WnB[...], preferred_element_type=_F32)


def _edge_step_body(el_ref, GA_ref, GB_ref, pS_ref, pR_ref, W1c, b1, W2, b2,
                    g, bb, elo, nel):
    el = el_ref[...]
    x = (jnp.dot(el, W1c[...], preferred_element_type=_F32)
         + _unpack2(GA_ref[0], pS_ref[...]) + _unpack2(GB_ref[0], pR_ref[...])
         + b1[...])
    h1 = jnp.maximum(x, 0.0)
    h2 = jnp.maximum(jnp.dot(h1, W2[...], preferred_element_type=_F32) + b2[...], 0.0)
    out = _ln(h2, g[...], bb[...])
    nel[...] = out
    elo[...] = el + out


def _node_step_body(nl_ref, a0_ref, a1_ref, Wa, Wb, b1, W2, b2, g, bb,
                    WnA, WnB, nlo, A, B):
    nl = nl_ref[...]
    agg = a0_ref[...] + a1_ref[...]
    x = (jnp.dot(nl, Wa[...], preferred_element_type=_F32)
         + jnp.dot(agg, Wb[...], preferred_element_type=_F32) + b1[...])
    h1 = jnp.maximum(x, 0.0)
    h2 = jnp.maximum(jnp.dot(h1, W2[...], preferred_element_type=_F32) + b2[...], 0.0)
    out = nl + _ln(h2, g[...], bb[...])
    nlo[...] = out
    A[...] = jnp.dot(out, WnA[...], preferred_element_type=_F32)
    B[...] = jnp.dot(out, WnB[...], preferred_element_type=_F32)


def _node_last_body(nl_ref, a0_ref, a1_ref, Wa, Wb, b1, W2, b2, g, bb,
                    dW1, db1, dW2, db2, dtp, dec):
    nl = nl_ref[...]
    agg = a0_ref[...] + a1_ref[...]
    x = (jnp.dot(nl, Wa[...], preferred_element_type=_F32)
         + jnp.dot(agg, Wb[...], preferred_element_type=_F32) + b1[...])
    h1 = jnp.maximum(x, 0.0)
    h2 = jnp.maximum(jnp.dot(h1, W2[...], preferred_element_type=_F32) + b2[...], 0.0)
    out = nl + _ln(h2, g[...], bb[...])
    h = jnp.dot(out, dW1[...], preferred_element_type=_F32) + db1[...]
    h = h * jax.nn.sigmoid(h)
    dec[...] = (jnp.dot(h, dW2[...], preferred_element_type=_F32) + db2[...]) * dtp[...]


def _row_spec(blk, w):
    return pl.BlockSpec((blk, w), lambda i: (i, 0))


def _full_spec(shape):
    nd = len(shape)
    return pl.BlockSpec(shape, lambda i: (0,) * nd)


def _part_spec(blk, part):
    return pl.BlockSpec((1, blk, L), lambda i, _p=part: (_p, i, 0))


def _tc_call(body, grid, in_arrays, blocked, out_shapes, out_blocked):
    in_specs = []
    for a, b in zip(in_arrays, blocked):
        if isinstance(b, pl.BlockSpec):
            in_specs.append(b)
        else:
            in_specs.append(_row_spec(*b) if b is not None else _full_spec(a.shape))
    out_specs = [_row_spec(*b) if b is not None else _full_spec(s.shape)
                 for s, b in zip(out_shapes, out_blocked)]
    return pl.pallas_call(
        body,
        grid=(grid,),
        in_specs=in_specs,
        out_specs=out_specs if len(out_specs) > 1 else out_specs[0],
        out_shape=out_shapes if len(out_shapes) > 1 else out_shapes[0],
    )(*in_arrays)


def _r1(v):
    return v.reshape(1, L)


def kernel(mesh_pos, node_type, u, load, senders, receivers,
           ne_W1, ne_b1, ne_W2, ne_b2, ne_g, ne_bb,
           ee_W1, ee_b1, ee_W2, ee_b2, ee_g, ee_bb,
           be_W1, be_b1, be_W2, be_b2, be_g, be_bb,
           bn_W1, bn_b1, bn_W2, bn_b2, bn_g, bn_bb,
           dec_W1, dec_b1, dec_W2, dec_b2):
    mesh_pos = mesh_pos[0]
    node_type = node_type[0]
    u = u[0]
    load = load[0]

    pad_e = E_PAD - E
    s_pad = jnp.concatenate([senders.astype(jnp.int32),
                             jnp.full((pad_e,), N, jnp.int32)])
    r_pad = jnp.concatenate([receivers.astype(jnp.int32),
                             jnp.full((pad_e,), N, jnp.int32)])
    s16 = (s_pad >> 1).reshape(16, CH_T, CHUNK)
    r16 = (r_pad >> 1).reshape(16, CH_T, CHUNK)
    pS = (s_pad & 1).astype(_F32).reshape(E_PAD, 1)
    pR = (r_pad & 1).astype(_F32).reshape(E_PAD, 1)
    r_w = r_pad.reshape(NWORK, CH_PER_W, CHUNK)

    T = jnp.zeros((N_PAD, L), _F32)
    T = T.at[:N, 0:3].set(mesh_pos).at[:N, 3:6].set(u)

    dparts = _sc_gather2(_pack2(T), _pack2(-T), s16, r16)

    W1d = jnp.zeros((L, L), _F32).at[0:6, :].set(ee_W1[0:6, :])
    w1n = ee_W1[6:7, :]
    el = _tc_call(
        _edge_encode_body, E_PAD // BLK_E,
        [dparts, dparts, pS, pR, W1d, w1n, _r1(ee_b1), ee_W2, _r1(ee_b2),
         _r1(ee_g), _r1(ee_bb)],
        [_part_spec(BLK_E, 0), _part_spec(BLK_E, 1), (BLK_E, 1), (BLK_E, 1),
         None, None, None, None, None, None, None],
        [jax.ShapeDtypeStruct((E_PAD, L), _F32)], [(BLK_E, L)])

    nf = jnp.zeros((N_PAD, 16), _F32)
    nf = nf.at[:N, 0:3].set(u).at[:N, 3:6].set(load).at[:N, 6:15].set(node_type)
    neW1p = jnp.zeros((16, L), _F32).at[0:15, :].set(ne_W1)
    nl, A, B = _tc_call(
        _node_encode_body, N_PAD // BLK_N,
        [nf, neW1p, _r1(ne_b1), ne_W2, _r1(ne_b2), _r1(ne_g), _r1(ne_bb),
         be_W1[0, 0:L, :], be_W1[0, L:2 * L, :]],
        [(BLK_N, 16)] + [None] * 8,
        [jax.ShapeDtypeStruct((N_PAD, L), _F32)] * 3,
        [(BLK_N, L)] * 3)

    zeros_n = jnp.zeros((N_PAD, L), _F32)

    for i in range(STEPS):
        Gp = _sc_gather2(_pack2(A), _pack2(B), s16, r16)
        el, new_el = _tc_call(
            _edge_step_body, E_PAD // BLK_E,
            [el, Gp, Gp, pS, pR, be_W1[i, 2 * L:3 * L, :], _r1(be_b1[i]),
             be_W2[i], _r1(be_b2[i]), _r1(be_g[i]), _r1(be_bb[i])],
            [(BLK_E, L), _part_spec(BLK_E, 0), _part_spec(BLK_E, 1),
             (BLK_E, 1), (BLK_E, 1), None, None, None, None, None, None],
            [jax.ShapeDtypeStruct((E_PAD, L), _F32)] * 2,
            [(BLK_E, L)] * 2)

        parts = _sc_scatter_add(new_el, r_w, zeros_n)
        ag0, ag1 = parts[0], parts[1]

        if i < STEPS - 1:
            nl, A, B = _tc_call(
                _node_step_body, N_PAD // BLK_N,
                [nl, ag0, ag1, bn_W1[i, 0:L, :], bn_W1[i, L:2 * L, :],
                 _r1(bn_b1[i]), bn_W2[i], _r1(bn_b2[i]), _r1(bn_g[i]),
                 _r1(bn_bb[i]), be_W1[i + 1, 0:L, :], be_W1[i + 1, L:2 * L, :]],
                [(BLK_N, L)] * 3 + [None] * 9,
                [jax.ShapeDtypeStruct((N_PAD, L), _F32)] * 3,
                [(BLK_N, L)] * 3)
        else:
            dW1p = jnp.zeros((L, L), _F32).at[:, 0:8].set(dec_W1)
            db1p = jnp.zeros((1, L), _F32).at[0, 0:8].set(dec_b1)
            dW2p = jnp.zeros((L, L), _F32).at[0:8, 0:TD * TW].set(dec_W2)
            db2p = jnp.zeros((1, L), _F32).at[0, 0:TD * TW].set(dec_b2)
            dt = jnp.repeat(jnp.arange(1, TW + 1, dtype=_F32), TD)
            dtp = jnp.zeros((1, L), _F32).at[0, 0:TD * TW].set(dt)
            dec = _tc_call(
                _node_last_body, N_PAD // BLK_N,
                [nl, ag0, ag1, bn_W1[i, 0:L, :], bn_W1[i, L:2 * L, :],
                 _r1(bn_b1[i]), bn_W2[i], _r1(bn_b2[i]), _r1(bn_g[i]),
                 _r1(bn_bb[i]), dW1p, db1p, dW2p, db2p, dtp],
                [(BLK_N, L)] * 3 + [None] * 12,
                [jax.ShapeDtypeStruct((N_PAD, L), _F32)], [(BLK_N, L)])

    return dec[:N, 0:TD * TW].reshape(N, TW, TD).transpose(1, 0, 2)
